# Initial kernel scaffold; baseline (speedup 1.0000x reference)
import jax
import jax.numpy as jnp
from jax.experimental import pallas as pl


def kernel(h, d, center):
    raise NotImplementedError("write your pallas kernel here")


# speedup vs baseline: 2.6852x; 2.6852x over previous
import functools

import jax
import jax.numpy as jnp
from jax import lax
from jax.experimental import pallas as pl
from jax.experimental.pallas import tpu as pltpu
from jax.experimental.pallas import tpu_sc as plsc

C = 8192
F = 256
B = 16384
ALPHA = 0.1

NC = 2
NS = 16
FH = F // NC
RPT = B // NS
G = 128
NG = RPT // G
CPT = C // NS
CNTR = C // 128


def _sc_segsum(h, d2):
    mesh = plsc.VectorSubcoreMesh(
        core_axis_name="c", subcore_axis_name="s", num_cores=NC,
        num_subcores=NS)

    @functools.partial(
        pl.kernel,
        out_type=(
            jax.ShapeDtypeStruct((NC, C, FH), jnp.float32),
            jax.ShapeDtypeStruct((CNTR, 128), jnp.float32),
        ),
        mesh=mesh,
        scratch_types=dict(
            seg_sh=pltpu.VMEM_SHARED((C, FH), jnp.float32),
            cnt_sh=pltpu.VMEM_SHARED((CNTR, 128), jnp.float32),
            zbuf=pltpu.VMEM((G, FH), jnp.float32),
            hbuf=pltpu.VMEM((G, FH), jnp.float32),
            idxbuf=pltpu.VMEM((NG, G), jnp.int32),
            cntloc=pltpu.VMEM((C,), jnp.float32),
            cnt2buf=pltpu.VMEM((CNTR, 128), jnp.float32),
            iotabuf=pltpu.VMEM((CNTR,), jnp.int32),
        ),
        compiler_params=pltpu.CompilerParams(needs_layout_passes=False),
    )
    def k(h_hbm, d_hbm, seg_hbm, cnt_hbm, seg_sh, cnt_sh, zbuf, hbuf,
          idxbuf, cntloc, cnt2buf, iotabuf):
        cid = lax.axis_index("c")
        sid = lax.axis_index("s")

        def fill(i, _):
            r = i // (FH // 16)
            col = (i % (FH // 16)) * 16
            zbuf[r, pl.ds(col, 16)] = jnp.zeros((16,), jnp.float32)
            return 0
        lax.fori_loop(0, G * (FH // 16), fill, 0)

        for kk in range(CPT // G):
            base = sid * CPT + kk * G
            pltpu.sync_copy(zbuf, seg_sh.at[pl.ds(base, G)])
        pltpu.sync_copy(zbuf.at[pl.ds(0, CNTR // NS)],
                        cnt_sh.at[pl.ds(sid * (CNTR // NS), CNTR // NS)])

        @pl.when(cid == 0)
        def _():
            def zc(i, _):
                cntloc[pl.ds(i * 16, 16)] = jnp.zeros((16,), jnp.float32)
                return 0
            lax.fori_loop(0, C // 16, zc, 0)
            for j in range(CNTR // 16):
                iotabuf[pl.ds(j * 16, 16)] = (
                    lax.iota(jnp.int32, 16) + j * 16)

        pltpu.sync_copy(d_hbm.at[pl.ds(sid * NG, NG)], idxbuf)
        plsc.subcore_barrier()

        for g in range(NG):
            row = sid * RPT + g * G
            pltpu.sync_copy(h_hbm.at[pl.ds(row, G), pl.ds(cid * FH, FH)],
                            hbuf)
            pltpu.sync_copy(hbuf, seg_sh.at[idxbuf.at[g]], add=True)

        @pl.when(cid == 0)
        def _():
            ones16 = jnp.ones((16,), jnp.float32)
            for g in range(NG):
                for j in range(G // 16):
                    idx16 = idxbuf[g, pl.ds(j * 16, 16)]
                    plsc.addupdate_scatter(cntloc, [idx16], ones16)

            def rp(i, _):
                r = i // 8
                col = (i % 8) * 16
                cnt2buf[r, pl.ds(col, 16)] = cntloc[pl.ds(i * 16, 16)]
                return 0
            lax.fori_loop(0, C // 16, rp, 0)
            pltpu.sync_copy(cnt2buf, cnt_sh.at[iotabuf], add=True)

        plsc.subcore_barrier()

        base = sid * CPT
        pltpu.sync_copy(seg_sh.at[pl.ds(base, CPT)],
                        seg_hbm.at[cid, pl.ds(base, CPT)])

        @pl.when(jnp.logical_and(cid == 0, sid == 0))
        def _():
            pltpu.sync_copy(cnt_sh, cnt_hbm)

    return k(h, d2)


HB = 512
CB = 256
NSTEP = C // CB


def _tc_body(h_ref, c_ref, seg_ref, cnt_ref, nc_ref, loss_ref):
    i = pl.program_id(0)
    hb = h_ref[...]
    cb = c_ref[...]
    segb = jnp.concatenate([seg_ref[0], seg_ref[1]], axis=1)
    cnt = cnt_ref[...]

    pos = cnt > 0.0
    denom = jnp.maximum(cnt, 1.0)
    diff = jnp.where(pos, segb / denom - cb, 0.0)
    nc_ref[...] = cb + ALPHA * diff

    part = (jnp.sum(hb * hb)
            - 2.0 * jnp.sum(segb * cb)
            + jnp.sum(cnt[:, 0] * jnp.sum(cb * cb, axis=1)))

    @pl.when(i == 0)
    def _():
        loss_ref[...] = jnp.zeros((1, 1), jnp.float32)

    loss_ref[...] += part.reshape(1, 1)


def _tc_combine(h, center, seg, cnt):
    return pl.pallas_call(
        _tc_body,
        grid=(NSTEP,),
        in_specs=[
            pl.BlockSpec((HB, F), lambda i: (i, 0)),
            pl.BlockSpec((CB, F), lambda i: (i, 0)),
            pl.BlockSpec((NC, CB, FH), lambda i: (0, i, 0)),
            pl.BlockSpec((CB, 1), lambda i: (i, 0)),
        ],
        out_specs=[
            pl.BlockSpec((CB, F), lambda i: (i, 0)),
            pl.BlockSpec((1, 1), lambda i: (0, 0)),
        ],
        out_shape=[
            jax.ShapeDtypeStruct((C, F), jnp.float32),
            jax.ShapeDtypeStruct((1, 1), jnp.float32),
        ],
    )(h, center, seg, cnt)


def kernel(h, d, center):
    d2 = d.astype(jnp.int32).reshape(B // 128, 128)
    seg, cnt = _sc_segsum(h, d2)
    cnt2d = cnt.reshape(C, 1)
    new_center, loss2d = _tc_combine(h, center, seg, cnt2d)
    loss = loss2d[0, 0] / (B * F)
    return loss, new_center


# speedup vs baseline: 3.4045x; 1.2679x over previous
import functools

import jax
import jax.numpy as jnp
from jax import lax
from jax.experimental import pallas as pl
from jax.experimental.pallas import tpu as pltpu
from jax.experimental.pallas import tpu_sc as plsc

C = 8192
F = 256
B = 16384
ALPHA = 0.1

NC = 2
NS = 16
FH = F // NC
RPT = B // NS
G = 128
NG = RPT // G
CPT = C // NS
CNTR = C // 128
ZR = 64


def _sc_segsum(h, d2):
    mesh = plsc.VectorSubcoreMesh(
        core_axis_name="c", subcore_axis_name="s", num_cores=NC,
        num_subcores=NS)

    @functools.partial(
        pl.kernel,
        out_type=(
            jax.ShapeDtypeStruct((NC, C, FH), jnp.float32),
            jax.ShapeDtypeStruct((CNTR, 128), jnp.float32),
        ),
        mesh=mesh,
        scratch_types=dict(
            seg_sh=pltpu.VMEM_SHARED((C, FH), jnp.float32),
            cnt_sh=pltpu.VMEM_SHARED((CNTR, 128), jnp.float32),
            zbuf=pltpu.VMEM((ZR, FH), jnp.float32),
            hbuf0=pltpu.VMEM((G, FH), jnp.float32),
            hbuf1=pltpu.VMEM((G, FH), jnp.float32),
            idxbuf=pltpu.VMEM((NG, G), jnp.int32),
            cntloc=pltpu.VMEM((C,), jnp.float32),
            cnt2buf=pltpu.VMEM((CNTR, 128), jnp.float32),
            iotabuf=pltpu.VMEM((CNTR,), jnp.int32),
            sem0=pltpu.SemaphoreType.DMA,
            sem1=pltpu.SemaphoreType.DMA,
        ),
        compiler_params=pltpu.CompilerParams(needs_layout_passes=False),
    )
    def k(h_hbm, d_hbm, seg_hbm, cnt_hbm, seg_sh, cnt_sh, zbuf, hbuf0,
          hbuf1, idxbuf, cntloc, cnt2buf, iotabuf, sem0, sem1):
        cid = lax.axis_index("c")
        sid = lax.axis_index("s")
        bufs = (hbuf0, hbuf1)
        sems = (sem0, sem1)

        idx_cp = pltpu.async_copy(d_hbm.at[pl.ds(sid * NG, NG)], idxbuf,
                                  sem1)

        def fill(i, _):
            for j in range(FH // 16):
                zbuf[i, pl.ds(j * 16, 16)] = jnp.zeros((16,), jnp.float32)
            return 0
        lax.fori_loop(0, ZR, fill, 0)

        for kk in range(CPT // ZR):
            base = sid * CPT + kk * ZR
            pltpu.sync_copy(zbuf, seg_sh.at[pl.ds(base, ZR)])
        pltpu.sync_copy(zbuf.at[pl.ds(0, CNTR // NS)],
                        cnt_sh.at[pl.ds(sid * (CNTR // NS), CNTR // NS)])

        @pl.when(cid == 0)
        def _():
            def zc(i, _):
                for j in range(8):
                    cntloc[pl.ds(i * 128 + j * 16, 16)] = jnp.zeros(
                        (16,), jnp.float32)
                return 0
            lax.fori_loop(0, C // 128, zc, 0)
            for j in range(CNTR // 16):
                iotabuf[pl.ds(j * 16, 16)] = (
                    lax.iota(jnp.int32, 16) + j * 16)

        idx_cp.wait()
        plsc.subcore_barrier()

        def h_load(g):
            row = sid * RPT + g * G
            return pltpu.async_copy(
                h_hbm.at[pl.ds(row, G), pl.ds(cid * FH, FH)],
                bufs[g % 2], sems[g % 2])

        cps = {0: h_load(0)}
        for g in range(NG):
            if g + 1 < NG:
                cps[g + 1] = h_load(g + 1)
            cps[g].wait()
            pltpu.sync_copy(bufs[g % 2], seg_sh.at[idxbuf.at[g]], add=True)

        @pl.when(cid == 0)
        def _():
            ones16 = jnp.ones((16,), jnp.float32)
            for g in range(NG):
                for j in range(G // 16):
                    idx16 = idxbuf[g, pl.ds(j * 16, 16)]
                    plsc.addupdate_scatter(cntloc, [idx16], ones16)

            def rp(i, _):
                for j in range(8):
                    cnt2buf[i, pl.ds(j * 16, 16)] = cntloc[
                        pl.ds(i * 128 + j * 16, 16)]
                return 0
            lax.fori_loop(0, CNTR, rp, 0)
            pltpu.sync_copy(cnt2buf, cnt_sh.at[iotabuf], add=True)

        plsc.subcore_barrier()

        base = sid * CPT
        pltpu.sync_copy(seg_sh.at[pl.ds(base, CPT)],
                        seg_hbm.at[cid, pl.ds(base, CPT)])

        @pl.when(jnp.logical_and(cid == 0, sid == 0))
        def _():
            pltpu.sync_copy(cnt_sh, cnt_hbm)

    return k(h, d2)


HB = 1024
CB = 256
NSTEP = C // CB


def _tc_sumsq_body(h_ref, o_ref):
    i = pl.program_id(0)
    hb = h_ref[...]

    @pl.when(i == 0)
    def _():
        o_ref[...] = jnp.zeros((1, 1), jnp.float32)

    o_ref[...] += jnp.sum(hb * hb).reshape(1, 1)


def _tc_sumsq(h):
    return pl.pallas_call(
        _tc_sumsq_body,
        grid=(B // HB,),
        in_specs=[pl.BlockSpec((HB, F), lambda i: (i, 0))],
        out_specs=pl.BlockSpec((1, 1), lambda i: (0, 0)),
        out_shape=jax.ShapeDtypeStruct((1, 1), jnp.float32),
    )(h)


def _tc_body(c_ref, seg_ref, cnt_ref, s2_ref, nc_ref, loss_ref):
    i = pl.program_id(0)
    cb = c_ref[...]
    segb = jnp.concatenate([seg_ref[0], seg_ref[1]], axis=1)

    eye = (lax.broadcasted_iota(jnp.int32, (128, 128), 0)
           == lax.broadcasted_iota(jnp.int32, (128, 128), 1)
           ).astype(jnp.float32)
    dn = (((1,), (1,)), ((), ()))
    cpair = cnt_ref[pl.ds(i * (CB // 128), CB // 128), :]
    c0 = lax.dot_general(eye, cpair[0:1, :], dn,
                         preferred_element_type=jnp.float32)
    c1 = lax.dot_general(eye, cpair[1:2, :], dn,
                         preferred_element_type=jnp.float32)
    cnt = jnp.concatenate([c0, c1], axis=0)

    pos = cnt > 0.0
    denom = jnp.maximum(cnt, 1.0)
    diff = jnp.where(pos, segb / denom - cb, 0.0)
    nc_ref[...] = cb + ALPHA * diff

    part = (- 2.0 * jnp.sum(segb * cb)
            + jnp.sum(cnt[:, 0] * jnp.sum(cb * cb, axis=1)))

    @pl.when(i == 0)
    def _():
        loss_ref[...] = s2_ref[...]

    loss_ref[...] += part.reshape(1, 1)


def _tc_combine(center, seg, cnt, sumh2):
    return pl.pallas_call(
        _tc_body,
        grid=(NSTEP,),
        in_specs=[
            pl.BlockSpec((CB, F), lambda i: (i, 0)),
            pl.BlockSpec((NC, CB, FH), lambda i: (0, i, 0)),
            pl.BlockSpec((CNTR, 128), lambda i: (0, 0)),
            pl.BlockSpec((1, 1), lambda i: (0, 0)),
        ],
        out_specs=[
            pl.BlockSpec((CB, F), lambda i: (i, 0)),
            pl.BlockSpec((1, 1), lambda i: (0, 0)),
        ],
        out_shape=[
            jax.ShapeDtypeStruct((C, F), jnp.float32),
            jax.ShapeDtypeStruct((1, 1), jnp.float32),
        ],
    )(center, seg, cnt, sumh2)


def kernel(h, d, center):
    d2 = d.astype(jnp.int32).reshape(B // 128, 128)
    seg, cnt = _sc_segsum(h, d2)
    sumh2 = _tc_sumsq(h)
    new_center, loss2d = _tc_combine(center, seg, cnt, sumh2)
    loss = loss2d[0, 0] / (B * F)
    return loss, new_center


# speedup vs baseline: 3.9462x; 1.1591x over previous
import functools

import jax
import jax.numpy as jnp
from jax import lax
from jax.experimental import pallas as pl
from jax.experimental.pallas import tpu as pltpu
from jax.experimental.pallas import tpu_sc as plsc

C = 8192
F = 256
B = 16384
ALPHA = 0.1

NC = 2
NS = 16
FH = F // NC
RPT = B // NS
G = 128
NG = RPT // G
CPT = C // NS
CNTR = C // 128
ZR = 64


def _sc_segsum(h, d2):
    mesh = plsc.VectorSubcoreMesh(
        core_axis_name="c", subcore_axis_name="s", num_cores=NC,
        num_subcores=NS)

    @functools.partial(
        pl.kernel,
        out_type=(
            jax.ShapeDtypeStruct((NC, C, FH), jnp.float32),
            jax.ShapeDtypeStruct((NC, CNTR, 128), jnp.float32),
        ),
        mesh=mesh,
        scratch_types=dict(
            seg_sh=pltpu.VMEM_SHARED((C, FH), jnp.float32),
            cnt_sh=pltpu.VMEM_SHARED((CNTR, 128), jnp.float32),
            zbuf=pltpu.VMEM((ZR, FH), jnp.float32),
            hbuf0=pltpu.VMEM((G, FH), jnp.float32),
            hbuf1=pltpu.VMEM((G, FH), jnp.float32),
            idxbuf=pltpu.VMEM((NG, G), jnp.int32),
            cntloc=pltpu.VMEM((C,), jnp.float32),
            cnt2buf=pltpu.VMEM((CNTR, 128), jnp.float32),
            iotabuf=pltpu.VMEM((CNTR,), jnp.int32),
            sem0=pltpu.SemaphoreType.DMA,
            sem1=pltpu.SemaphoreType.DMA,
            sem2=pltpu.SemaphoreType.DMA,
        ),
        compiler_params=pltpu.CompilerParams(needs_layout_passes=False),
    )
    def k(h_hbm, d_hbm, seg_hbm, cnt_hbm, seg_sh, cnt_sh, zbuf, hbuf0,
          hbuf1, idxbuf, cntloc, cnt2buf, iotabuf, sem0, sem1, sem2):
        cid = lax.axis_index("c")
        sid = lax.axis_index("s")
        bufs = (hbuf0, hbuf1)
        sems = (sem0, sem1)

        def h_load(g):
            row = sid * RPT + g * G
            return pltpu.async_copy(
                h_hbm.at[pl.ds(row, G), pl.ds(cid * FH, FH)],
                bufs[g % 2], sems[g % 2])

        cps = {0: h_load(0), 1: h_load(1)}
        idx_cp = pltpu.async_copy(d_hbm.at[pl.ds(sid * NG, NG)], idxbuf,
                                  sem2)

        def fill(i, _):
            for j in range(FH // 16):
                zbuf[i, pl.ds(j * 16, 16)] = jnp.zeros((16,), jnp.float32)
            return 0
        lax.fori_loop(0, ZR, fill, 0)

        for kk in range(CPT // ZR):
            base = sid * CPT + kk * ZR
            pltpu.sync_copy(zbuf, seg_sh.at[pl.ds(base, ZR)])
        pltpu.sync_copy(zbuf.at[pl.ds(0, CNTR // NS)],
                        cnt_sh.at[pl.ds(sid * (CNTR // NS), CNTR // NS)])

        def zc(i, _):
            for j in range(8):
                cntloc[pl.ds(i * 128 + j * 16, 16)] = jnp.zeros(
                    (16,), jnp.float32)
            return 0
        lax.fori_loop(0, C // 128, zc, 0)
        for j in range(CNTR // 16):
            iotabuf[pl.ds(j * 16, 16)] = lax.iota(jnp.int32, 16) + j * 16

        idx_cp.wait()
        plsc.subcore_barrier()

        for g in range(NG):
            cps[g].wait()
            pltpu.sync_copy(bufs[g % 2], seg_sh.at[idxbuf.at[g]], add=True)
            if g + 2 < NG:
                cps[g + 2] = h_load(g + 2)

        ones16 = jnp.ones((16,), jnp.float32)

        def count_groups(g_lo):
            def _():
                for g in range(g_lo, g_lo + NG // 2):
                    for j in range(G // 16):
                        idx16 = idxbuf[g, pl.ds(j * 16, 16)]
                        plsc.addupdate_scatter(cntloc, [idx16], ones16)
            return _

        pl.when(cid == 0)(count_groups(0))
        pl.when(cid == 1)(count_groups(NG // 2))

        def rp(i, _):
            for j in range(8):
                cnt2buf[i, pl.ds(j * 16, 16)] = cntloc[
                    pl.ds(i * 128 + j * 16, 16)]
            return 0
        lax.fori_loop(0, CNTR, rp, 0)
        pltpu.sync_copy(cnt2buf, cnt_sh.at[iotabuf], add=True)

        plsc.subcore_barrier()

        base = sid * CPT
        pltpu.sync_copy(seg_sh.at[pl.ds(base, CPT)],
                        seg_hbm.at[cid, pl.ds(base, CPT)])

        @pl.when(sid == 0)
        def _():
            pltpu.sync_copy(cnt_sh, cnt_hbm.at[cid])

    return k(h, d2)


HB = 2048
CB = 512
NSTEP = C // CB


def _tc_sumsq_body(h_ref, o_ref):
    i = pl.program_id(0)
    hb = h_ref[...]

    @pl.when(i == 0)
    def _():
        o_ref[...] = jnp.zeros((1, 1), jnp.float32)

    o_ref[...] += jnp.sum(hb * hb).reshape(1, 1)


def _tc_sumsq(h):
    return pl.pallas_call(
        _tc_sumsq_body,
        grid=(B // HB,),
        in_specs=[pl.BlockSpec((HB, F), lambda i: (i, 0))],
        out_specs=pl.BlockSpec((1, 1), lambda i: (0, 0)),
        out_shape=jax.ShapeDtypeStruct((1, 1), jnp.float32),
    )(h)


def _tc_body(c_ref, seg_ref, cnt_ref, s2_ref, nc_ref, loss_ref):
    i = pl.program_id(0)
    cb = c_ref[...]
    segb = jnp.concatenate([seg_ref[0], seg_ref[1]], axis=1)

    eye = (lax.broadcasted_iota(jnp.int32, (128, 128), 0)
           == lax.broadcasted_iota(jnp.int32, (128, 128), 1)
           ).astype(jnp.float32)
    dn = (((1,), (1,)), ((), ()))
    nrow = CB // 128
    cpair = (cnt_ref[0, pl.ds(i * nrow, nrow), :]
             + cnt_ref[1, pl.ds(i * nrow, nrow), :])
    cnt = jnp.concatenate(
        [lax.dot_general(eye, cpair[j:j + 1, :], dn,
                         preferred_element_type=jnp.float32)
         for j in range(nrow)], axis=0)

    pos = cnt > 0.0
    denom = jnp.maximum(cnt, 1.0)
    diff = jnp.where(pos, segb / denom - cb, 0.0)
    nc_ref[...] = cb + ALPHA * diff

    part = (- 2.0 * jnp.sum(segb * cb)
            + jnp.sum(cnt[:, 0] * jnp.sum(cb * cb, axis=1)))

    @pl.when(i == 0)
    def _():
        loss_ref[...] = s2_ref[...]

    loss_ref[...] += part.reshape(1, 1)


def _tc_combine(center, seg, cnt, sumh2):
    return pl.pallas_call(
        _tc_body,
        grid=(NSTEP,),
        in_specs=[
            pl.BlockSpec((CB, F), lambda i: (i, 0)),
            pl.BlockSpec((NC, CB, FH), lambda i: (0, i, 0)),
            pl.BlockSpec((NC, CNTR, 128), lambda i: (0, 0, 0)),
            pl.BlockSpec((1, 1), lambda i: (0, 0)),
        ],
        out_specs=[
            pl.BlockSpec((CB, F), lambda i: (i, 0)),
            pl.BlockSpec((1, 1), lambda i: (0, 0)),
        ],
        out_shape=[
            jax.ShapeDtypeStruct((C, F), jnp.float32),
            jax.ShapeDtypeStruct((1, 1), jnp.float32),
        ],
    )(center, seg, cnt, sumh2)


def kernel(h, d, center):
    d2 = d.astype(jnp.int32).reshape(B // 128, 128)
    seg, cnt = _sc_segsum(h, d2)
    sumh2 = _tc_sumsq(h)
    new_center, loss2d = _tc_combine(center, seg, cnt, sumh2)
    loss = loss2d[0, 0] / (B * F)
    return loss, new_center


# speedup vs baseline: 4.1527x; 1.0523x over previous
import functools

import jax
import jax.numpy as jnp
from jax import lax
from jax.experimental import pallas as pl
from jax.experimental.pallas import tpu as pltpu
from jax.experimental.pallas import tpu_sc as plsc

C = 8192
F = 256
B = 16384
ALPHA = 0.1

NC = 2
NS = 16
FH = F // NC
RPT = B // NS
G = 128
NG = RPT // G
CPT = C // NS
CNTR = C // 128
ZR = 64
CC = 128
NCH = CPT // CC


def _sc_center(h, d2, center):
    mesh = plsc.VectorSubcoreMesh(
        core_axis_name="c", subcore_axis_name="s", num_cores=NC,
        num_subcores=NS)

    @functools.partial(
        pl.kernel,
        out_type=(
            jax.ShapeDtypeStruct((C, F), jnp.float32),
            jax.ShapeDtypeStruct((NC, NS, 16), jnp.float32),
            jax.ShapeDtypeStruct((NC, NS, 16), jnp.float32),
        ),
        mesh=mesh,
        scratch_types=dict(
            seg_sh=pltpu.VMEM_SHARED((C, FH), jnp.float32),
            cnt_sh=pltpu.VMEM_SHARED((CNTR, 128), jnp.float32),
            zbuf=pltpu.VMEM((ZR, FH), jnp.float32),
            hbuf0=pltpu.VMEM((G, FH), jnp.float32),
            hbuf1=pltpu.VMEM((G, FH), jnp.float32),
            idxbuf=pltpu.VMEM((NG, G), jnp.int32),
            cntloc=pltpu.VMEM((C,), jnp.float32),
            cnt2buf=pltpu.VMEM((CNTR, 128), jnp.float32),
            iotabuf=pltpu.VMEM((CNTR,), jnp.int32),
            facbuf=pltpu.VMEM((CPT,), jnp.float32),
            gbuf=pltpu.VMEM((CPT,), jnp.float32),
            cflat=pltpu.VMEM((CPT,), jnp.float32),
            pbuf=pltpu.VMEM((2, 16), jnp.float32),
            sem0=pltpu.SemaphoreType.DMA,
            sem1=pltpu.SemaphoreType.DMA,
            sem2=pltpu.SemaphoreType.DMA,
        ),
        compiler_params=pltpu.CompilerParams(needs_layout_passes=False),
    )
    def k(h_hbm, d_hbm, c_hbm, nc_hbm, pdot_hbm, pcn_hbm, seg_sh, cnt_sh, zbuf,
          hbuf0, hbuf1, idxbuf, cntloc, cnt2buf, iotabuf, facbuf, gbuf,
          cflat, pbuf, sem0, sem1, sem2):
        cid = lax.axis_index("c")
        sid = lax.axis_index("s")
        bufs = (hbuf0, hbuf1)
        sems = (sem0, sem1)

        def h_load(g):
            row = sid * RPT + g * G
            return pltpu.async_copy(
                h_hbm.at[pl.ds(row, G), pl.ds(cid * FH, FH)],
                bufs[g % 2], sems[g % 2])

        cps = {0: h_load(0), 1: h_load(1)}
        idx_cp = pltpu.async_copy(d_hbm.at[pl.ds(sid * NG, NG)], idxbuf,
                                  sem2)

        def fill(i, _):
            for j in range(FH // 16):
                zbuf[i, pl.ds(j * 16, 16)] = jnp.zeros((16,), jnp.float32)
            return 0
        lax.fori_loop(0, ZR, fill, 0)

        for kk in range(CPT // ZR):
            base = sid * CPT + kk * ZR
            pltpu.sync_copy(zbuf, seg_sh.at[pl.ds(base, ZR)])
        pltpu.sync_copy(zbuf.at[pl.ds(0, CNTR // NS)],
                        cnt_sh.at[pl.ds(sid * (CNTR // NS), CNTR // NS)])

        def zc(i, _):
            for j in range(8):
                cntloc[pl.ds(i * 128 + j * 16, 16)] = jnp.zeros(
                    (16,), jnp.float32)
            return 0
        lax.fori_loop(0, C // 128, zc, 0)
        for j in range(CNTR // 16):
            iotabuf[pl.ds(j * 16, 16)] = lax.iota(jnp.int32, 16) + j * 16

        idx_cp.wait()
        plsc.subcore_barrier()

        for g in range(NG):
            cps[g].wait()
            pltpu.sync_copy(bufs[g % 2], seg_sh.at[idxbuf.at[g]], add=True)
            if g + 2 < NG:
                cps[g + 2] = h_load(g + 2)

        ones16 = jnp.ones((16,), jnp.float32)
        for g in range(NG):
            for j in range(G // 16):
                idx16 = idxbuf[g, pl.ds(j * 16, 16)]
                plsc.addupdate_scatter(cntloc, [idx16], ones16)

        def rp(i, _):
            for j in range(8):
                cnt2buf[i, pl.ds(j * 16, 16)] = cntloc[
                    pl.ds(i * 128 + j * 16, 16)]
            return 0
        lax.fori_loop(0, CNTR, rp, 0)
        pltpu.sync_copy(cnt2buf, cnt_sh.at[iotabuf], add=True)

        plsc.subcore_barrier()

        pltpu.sync_copy(cnt_sh.at[pl.ds(sid * (CPT // 128), CPT // 128)],
                        cnt2buf.at[pl.ds(0, CPT // 128)])

        def mkfac(i, _):
            cv = cnt2buf[i // 8, pl.ds((i % 8) * 16, 16)]
            pos = cv > 0.0
            den = jnp.maximum(cv, 1.0)
            facbuf[pl.ds(i * 16, 16)] = jnp.where(pos, ALPHA / den, 0.0)
            gbuf[pl.ds(i * 16, 16)] = jnp.where(pos, 1.0 - ALPHA, 1.0)
            cflat[pl.ds(i * 16, 16)] = cv
            return 0
        lax.fori_loop(0, CPT // 16, mkfac, 0)

        acc_dot = jnp.zeros((16,), jnp.float32)
        acc_cn = jnp.zeros((16,), jnp.float32)
        for cc in range(NCH):
            kbase = sid * CPT + cc * CC
            c_cp = pltpu.async_copy(
                c_hbm.at[pl.ds(kbase, CC), pl.ds(cid * FH, FH)], hbuf0,
                sem0)
            pltpu.sync_copy(seg_sh.at[pl.ds(kbase, CC)], hbuf1)
            c_cp.wait()

            def upd(bi, acc):
                ad, an = acc
                base = cc * CC + bi * 16
                fvec = facbuf[pl.ds(base, 16)]
                gvec = gbuf[pl.ds(base, 16)]
                nvec = cflat[pl.ds(base, 16)]
                for t in range(16):
                    i = bi * 16 + t
                    fk = fvec[t]
                    gk = gvec[t]
                    nk = nvec[t]
                    for j in range(FH // 16):
                        cv = hbuf0[i, pl.ds(j * 16, 16)]
                        sv = hbuf1[i, pl.ds(j * 16, 16)]
                        hbuf1[i, pl.ds(j * 16, 16)] = gk * cv + fk * sv
                        ad = ad + sv * cv
                        an = an + (nk * cv) * cv
                return (ad, an)
            acc_dot, acc_cn = lax.fori_loop(0, CC // 16, upd,
                                            (acc_dot, acc_cn))
            pltpu.sync_copy(hbuf1,
                            nc_hbm.at[pl.ds(kbase, CC),
                                      pl.ds(cid * FH, FH)])

        pbuf[0, pl.ds(0, 16)] = acc_dot
        pbuf[1, pl.ds(0, 16)] = acc_cn
        pltpu.sync_copy(pbuf.at[0], pdot_hbm.at[cid, sid])
        pltpu.sync_copy(pbuf.at[1], pcn_hbm.at[cid, sid])

    return k(h, d2, center)


HB = 2048


def _tc_sumsq_body(h_ref, o_ref):
    i = pl.program_id(0)
    hb = h_ref[...]

    @pl.when(i == 0)
    def _():
        o_ref[...] = jnp.zeros((1, 1), jnp.float32)

    o_ref[...] += jnp.sum(hb * hb).reshape(1, 1)


def _tc_sumsq(h):
    return pl.pallas_call(
        _tc_sumsq_body,
        grid=(B // HB,),
        in_specs=[pl.BlockSpec((HB, F), lambda i: (i, 0))],
        out_specs=pl.BlockSpec((1, 1), lambda i: (0, 0)),
        out_shape=jax.ShapeDtypeStruct((1, 1), jnp.float32),
    )(h)


def _tc_finish_body(s2_ref, pd_ref, pc_ref, loss_ref):
    loss = (s2_ref[0, 0] - 2.0 * jnp.sum(pd_ref[...])
            + jnp.sum(pc_ref[...])) / (B * F)
    loss_ref[...] = loss.reshape(1, 1)


def _tc_finish(sumh2, pdot, pcn):
    return pl.pallas_call(
        _tc_finish_body,
        out_shape=jax.ShapeDtypeStruct((1, 1), jnp.float32),
    )(sumh2, pdot, pcn)


def kernel(h, d, center):
    d2 = d.astype(jnp.int32).reshape(B // 128, 128)
    new_center, pdot, pcn = _sc_center(h, d2, center)
    sumh2 = _tc_sumsq(h)
    loss2d = _tc_finish(sumh2, pdot, pcn)
    return loss2d[0, 0], new_center
